# baseline (speedup 1.0000x reference)
import functools

import jax
import jax.numpy as jnp
from jax import lax
from jax.experimental import pallas as pl
from jax.experimental.pallas import tpu as pltpu
from jax.experimental.pallas import tpu_sc as plsc

N = 10000
E = 320000
F_IN = 128
HID = 8
HEADS = 8
C = 40

NC = 2
NS = 16
LANES = 16
CH = 128
CPT = 81
E_PAD = NC * NS * CPT * CH
N_PAD = 10240
ROWS_PER_TILE = N_PAD // NS
PAD_DST = N

ROW1 = 80
ROW2 = 48

BLK = 1024
GRID = N_PAD // BLK


def _tc_prep1_body(x_ref, w1_ref, a1_ref, table_ref, adtab_ref):
    h = jnp.dot(x_ref[...], w1_ref[...], precision=lax.Precision.HIGHEST,
                preferred_element_type=jnp.float32)
    aa = jnp.dot(h, a1_ref[...], precision=lax.Precision.HIGHEST,
                 preferred_element_type=jnp.float32)
    z = jnp.zeros((BLK, 8), jnp.float32)
    table_ref[...] = jnp.concatenate([h, aa[:, 0:8], z], axis=1)
    adtab_ref[...] = jnp.concatenate([aa[:, 8:16], z], axis=1)


def _tc_prep1(x, W1, A1):
    return pl.pallas_call(
        _tc_prep1_body,
        grid=(GRID,),
        in_specs=[
            pl.BlockSpec((BLK, F_IN), lambda i: (i, 0)),
            pl.BlockSpec((F_IN, HEADS * HID), lambda i: (0, 0)),
            pl.BlockSpec((HEADS * HID, 16), lambda i: (0, 0)),
        ],
        out_specs=[
            pl.BlockSpec((BLK, ROW1), lambda i: (i, 0)),
            pl.BlockSpec((BLK, 16), lambda i: (i, 0)),
        ],
        out_shape=[
            jax.ShapeDtypeStruct((N_PAD, ROW1), jnp.float32),
            jax.ShapeDtypeStruct((N_PAD, 16), jnp.float32),
        ],
    )(x, W1, A1)


def _tc_mid_body(acc_ref, b1_ref, w2_ref, a2_ref, p_ref, table_ref, adtab_ref):
    u = acc_ref[0, :, 0:64] + acc_ref[1, :, 0:64]
    den = acc_ref[0, :, 64:72] + acc_ref[1, :, 64:72]
    r = 1.0 / (den + 1e-16)
    rex = jnp.dot(r, p_ref[...], precision=lax.Precision.HIGHEST,
                  preferred_element_type=jnp.float32)
    out1 = u * rex + b1_ref[...]
    hmid = jnp.where(out1 > 0, out1, jnp.exp(jnp.minimum(out1, 0.0)) - 1.0)
    h2 = jnp.dot(hmid, w2_ref[...], precision=lax.Precision.HIGHEST,
                 preferred_element_type=jnp.float32)
    aa2 = jnp.dot(h2, a2_ref[...], precision=lax.Precision.HIGHEST,
                  preferred_element_type=jnp.float32)
    table_ref[...] = jnp.concatenate(
        [h2, aa2[:, 0:1], jnp.zeros((BLK, 7), jnp.float32)], axis=1)
    adtab_ref[...] = jnp.concatenate(
        [aa2[:, 1:2], jnp.zeros((BLK, 15), jnp.float32)], axis=1)


def _tc_mid(acc1, b1, W2, a2, P):
    return pl.pallas_call(
        _tc_mid_body,
        grid=(GRID,),
        in_specs=[
            pl.BlockSpec((2, BLK, ROW1), lambda i: (0, i, 0)),
            pl.BlockSpec((1, HEADS * HID), lambda i: (0, 0)),
            pl.BlockSpec((HEADS * HID, C), lambda i: (0, 0)),
            pl.BlockSpec((C, 2), lambda i: (0, 0)),
            pl.BlockSpec((HEADS, HEADS * HID), lambda i: (0, 0)),
        ],
        out_specs=[
            pl.BlockSpec((BLK, ROW2), lambda i: (i, 0)),
            pl.BlockSpec((BLK, 16), lambda i: (i, 0)),
        ],
        out_shape=[
            jax.ShapeDtypeStruct((N_PAD, ROW2), jnp.float32),
            jax.ShapeDtypeStruct((N_PAD, 16), jnp.float32),
        ],
    )(acc1, b1, W2, a2, P)


def _tc_final_body(acc_ref, b2_ref, out_ref):
    u = acc_ref[0, :, 0:40] + acc_ref[1, :, 0:40]
    den = acc_ref[0, :, 40:41] + acc_ref[1, :, 40:41]
    out_ref[...] = u / (den + 1e-16) + b2_ref[...]


def _tc_final(acc2, b2):
    return pl.pallas_call(
        _tc_final_body,
        grid=(GRID,),
        in_specs=[
            pl.BlockSpec((2, BLK, ROW2), lambda i: (0, i, 0)),
            pl.BlockSpec((1, C), lambda i: (0, 0)),
        ],
        out_specs=pl.BlockSpec((BLK, C), lambda i: (i, 0)),
        out_shape=jax.ShapeDtypeStruct((N_PAD, C), jnp.float32),
    )(acc2, b2)


def _sc_edge_pass(layer, row_w):
    mesh = plsc.VectorSubcoreMesh(
        core_axis_name="c", subcore_axis_name="s", num_cores=NC)
    NBUF = 3
    UNROLL = 4

    def body(table_hbm, adtab_hbm, src_hbm, dst_hbm, zeros_hbm, out_hbm,
             srci, dsti, rows0, rows1, rows2, adr0, adr1, adr2, acc,
             gs0, gs1, gs2, as0, as1, as2, ss0, ss1, ss2):
        rows = (rows0, rows1, rows2)
        adr = (adr0, adr1, adr2)
        gsem = (gs0, gs1, gs2)
        asem = (as0, as1, as2)
        ssem = (ss0, ss1, ss2)

        cid = lax.axis_index("c")
        sid = lax.axis_index("s")
        wid = sid * NC + cid

        r0 = sid * ROWS_PER_TILE
        pltpu.sync_copy(zeros_hbm.at[pl.ds(r0, ROWS_PER_TILE)],
                        acc.at[pl.ds(r0, ROWS_PER_TILE)])

        pltpu.sync_copy(src_hbm.at[pl.ds(wid * CPT, CPT)], srci)
        pltpu.sync_copy(dst_hbm.at[pl.ds(wid * CPT, CPT)], dsti)
        plsc.subcore_barrier()

        iota = lax.iota(jnp.int32, LANES)
        hi = lax.shift_right_logical(iota, 3)

        def fire_gathers(j, b):
            pltpu.async_copy(table_hbm.at[srci.at[j]], rows[b], gsem[b])
            pltpu.async_copy(adtab_hbm.at[dsti.at[j]], adr[b], asem[b])

        def wait_gathers(j, b):
            pltpu.make_async_copy(
                table_hbm.at[srci.at[j]], rows[b], gsem[b]).wait()
            pltpu.make_async_copy(
                adtab_hbm.at[dsti.at[j]], adr[b], asem[b]).wait()

        def fire_scatter(j, b):
            pltpu.async_copy(rows[b], acc.at[dsti.at[j]], ssem[b], add=True)

        def wait_scatter(j, b):
            pltpu.make_async_copy(
                rows[b], acc.at[dsti.at[j]], ssem[b]).wait()

        if layer == 1:
            def edge_one(b, k):
                as16 = rows[b][k, pl.ds(64, LANES)]
                ad16 = adr[b][k, :]
                e = as16 + ad16
                s = jnp.exp(jnp.maximum(e, 0.2 * e))
                rows[b][k, pl.ds(64, LANES)] = s
                for v in range(4):
                    m = jnp.take_along_axis(s, hi + (2 * v), axis=0)
                    rows[b][k, pl.ds(16 * v, LANES)] = (
                        rows[b][k, pl.ds(16 * v, LANES)] * m)
        else:
            def edge_one(b, k):
                r2 = rows[b][k, pl.ds(32, LANES)]
                adv = adr[b][k, :]
                as_b = jnp.take_along_axis(
                    r2, jnp.full((LANES,), 8, jnp.int32), axis=0)
                ad_b = jnp.take_along_axis(
                    adv, jnp.zeros((LANES,), jnp.int32), axis=0)
                e = as_b + ad_b
                s = jnp.exp(jnp.maximum(e, 0.2 * e))
                rows[b][k, pl.ds(0, LANES)] = rows[b][k, pl.ds(0, LANES)] * s
                rows[b][k, pl.ds(16, LANES)] = rows[b][k, pl.ds(16, LANES)] * s
                rows[b][k, pl.ds(32, LANES)] = jnp.where(iota == 8, s, r2 * s)

        def compute(b):
            @functools.partial(plsc.parallel_loop, 0, CH, unroll=UNROLL)
            def _(k):
                edge_one(b, k)

        fire_gathers(0, 0)

        def step(t, _):
            for u in range(NBUF):
                j = t * NBUF + u
                b = u
                wait_gathers(j, b)
                bn = (u + 1) % NBUF

                @pl.when(j + 1 < CPT)
                def _():
                    @pl.when(j >= 2)
                    def _():
                        wait_scatter(j - 2, bn)
                    fire_gathers(j + 1, bn)

                compute(b)
                fire_scatter(j, b)
            return ()

        lax.fori_loop(0, CPT // NBUF, step, ())

        wait_scatter(CPT - 3, (CPT - 3) % NBUF)
        wait_scatter(CPT - 2, (CPT - 2) % NBUF)
        wait_scatter(CPT - 1, (CPT - 1) % NBUF)
        plsc.subcore_barrier()
        pltpu.sync_copy(acc.at[pl.ds(r0, ROWS_PER_TILE)],
                        out_hbm.at[cid, pl.ds(r0, ROWS_PER_TILE)])

    return pl.kernel(
        body,
        out_type=jax.ShapeDtypeStruct((NC, N_PAD, row_w), jnp.float32),
        mesh=mesh,
        compiler_params=pltpu.CompilerParams(use_tc_tiling_on_sc=False),
        scratch_types=[
            pltpu.VMEM((CPT, CH), jnp.int32),
            pltpu.VMEM((CPT, CH), jnp.int32),
            pltpu.VMEM((CH, row_w), jnp.float32),
            pltpu.VMEM((CH, row_w), jnp.float32),
            pltpu.VMEM((CH, row_w), jnp.float32),
            pltpu.VMEM((CH, 16), jnp.float32),
            pltpu.VMEM((CH, 16), jnp.float32),
            pltpu.VMEM((CH, 16), jnp.float32),
            pltpu.VMEM_SHARED((N_PAD, row_w), jnp.float32),
            pltpu.SemaphoreType.DMA,
            pltpu.SemaphoreType.DMA,
            pltpu.SemaphoreType.DMA,
            pltpu.SemaphoreType.DMA,
            pltpu.SemaphoreType.DMA,
            pltpu.SemaphoreType.DMA,
            pltpu.SemaphoreType.DMA,
            pltpu.SemaphoreType.DMA,
            pltpu.SemaphoreType.DMA,
        ],
    )


def kernel(x, edge_index, W1, a_src1, a_dst1, b1, W2, a_src2, a_dst2, b2, Q,
           epoch):
    src = edge_index[0]
    dst = edge_index[1]
    pad = E_PAD - E
    src = jnp.concatenate([src, jnp.full((pad,), PAD_DST, jnp.int32)])
    dst = jnp.concatenate([dst, jnp.full((pad,), PAD_DST, jnp.int32)])
    src2d = src.reshape(E_PAD // CH, CH)
    dst2d = dst.reshape(E_PAD // CH, CH)
    xp = jnp.concatenate([x, jnp.zeros((N_PAD - N, F_IN), jnp.float32)])

    eye = jnp.eye(HEADS, dtype=jnp.float32)
    Asrc1 = (a_src1[:, :, None] * eye[:, None, :]).reshape(HEADS * HID, HEADS)
    Adst1 = (a_dst1[:, :, None] * eye[:, None, :]).reshape(HEADS * HID, HEADS)
    A1 = jnp.concatenate([Asrc1, Adst1], axis=1)
    a2 = jnp.concatenate([a_src2.T, a_dst2.T], axis=1)
    P = jnp.repeat(eye, HID, axis=1)

    table1, adtab1 = _tc_prep1(xp, W1, A1)
    zeros1 = jnp.zeros((N_PAD, ROW1), jnp.float32)
    acc1 = _sc_edge_pass(1, ROW1)(table1, adtab1, src2d, dst2d, zeros1)
    table2, adtab2 = _tc_mid(acc1, b1.reshape(1, -1), W2, a2, P)
    zeros2 = jnp.zeros((N_PAD, ROW2), jnp.float32)
    acc2 = _sc_edge_pass(2, ROW2)(table2, adtab2, src2d, dst2d, zeros2)
    out = _tc_final(acc2, b2.reshape(1, -1))
    return (out[:N], Q)


# speedup vs baseline: 1.1886x; 1.1886x over previous
import functools

import jax
import jax.numpy as jnp
from jax import lax
from jax.experimental import pallas as pl
from jax.experimental.pallas import tpu as pltpu
from jax.experimental.pallas import tpu_sc as plsc

N = 10000
E = 320000
F_IN = 128
HID = 8
HEADS = 8
C = 40

NC = 2
NS = 16
LANES = 16
CH = 128
CPT = 80
E_PAD = NC * NS * CPT * CH
N_PAD = 10240
ROWS_PER_TILE = N_PAD // NS
PAD_DST = N

ROW1 = 80
ROW2 = 48

BLK = 1024
GRID = N_PAD // BLK


def _tc_prep1_body(x_ref, w1_ref, a1_ref, table_ref, adtab_ref):
    h = jnp.dot(x_ref[...], w1_ref[...], precision=lax.Precision.HIGHEST,
                preferred_element_type=jnp.float32)
    aa = jnp.dot(h, a1_ref[...], precision=lax.Precision.HIGHEST,
                 preferred_element_type=jnp.float32)
    z = jnp.zeros((BLK, 8), jnp.float32)
    table_ref[...] = jnp.concatenate([h, aa[:, 0:8], z], axis=1)
    adtab_ref[...] = jnp.concatenate([aa[:, 8:16], z], axis=1)


def _tc_prep1(x, W1, A1):
    return pl.pallas_call(
        _tc_prep1_body,
        grid=(GRID,),
        in_specs=[
            pl.BlockSpec((BLK, F_IN), lambda i: (i, 0)),
            pl.BlockSpec((F_IN, HEADS * HID), lambda i: (0, 0)),
            pl.BlockSpec((HEADS * HID, 16), lambda i: (0, 0)),
        ],
        out_specs=[
            pl.BlockSpec((BLK, ROW1), lambda i: (i, 0)),
            pl.BlockSpec((BLK, 16), lambda i: (i, 0)),
        ],
        out_shape=[
            jax.ShapeDtypeStruct((N_PAD, ROW1), jnp.float32),
            jax.ShapeDtypeStruct((N_PAD, 16), jnp.float32),
        ],
    )(x, W1, A1)


def _tc_mid_body(acc_ref, b1_ref, w2_ref, a2_ref, p_ref, table_ref, adtab_ref):
    u = acc_ref[0, :, 0:64] + acc_ref[1, :, 0:64]
    den = acc_ref[0, :, 64:72] + acc_ref[1, :, 64:72]
    r = 1.0 / (den + 1e-16)
    rex = jnp.dot(r, p_ref[...], precision=lax.Precision.HIGHEST,
                  preferred_element_type=jnp.float32)
    out1 = u * rex + b1_ref[...]
    hmid = jnp.where(out1 > 0, out1, jnp.exp(jnp.minimum(out1, 0.0)) - 1.0)
    h2 = jnp.dot(hmid, w2_ref[...], precision=lax.Precision.HIGHEST,
                 preferred_element_type=jnp.float32)
    aa2 = jnp.dot(h2, a2_ref[...], precision=lax.Precision.HIGHEST,
                  preferred_element_type=jnp.float32)
    table_ref[...] = jnp.concatenate(
        [h2, aa2[:, 0:1], jnp.zeros((BLK, 7), jnp.float32)], axis=1)
    adtab_ref[...] = jnp.concatenate(
        [aa2[:, 1:2], jnp.zeros((BLK, 15), jnp.float32)], axis=1)


def _tc_mid(acc1, b1, W2, a2, P):
    return pl.pallas_call(
        _tc_mid_body,
        grid=(GRID,),
        in_specs=[
            pl.BlockSpec((2, BLK, ROW1), lambda i: (0, i, 0)),
            pl.BlockSpec((1, HEADS * HID), lambda i: (0, 0)),
            pl.BlockSpec((HEADS * HID, C), lambda i: (0, 0)),
            pl.BlockSpec((C, 2), lambda i: (0, 0)),
            pl.BlockSpec((HEADS, HEADS * HID), lambda i: (0, 0)),
        ],
        out_specs=[
            pl.BlockSpec((BLK, ROW2), lambda i: (i, 0)),
            pl.BlockSpec((BLK, 16), lambda i: (i, 0)),
        ],
        out_shape=[
            jax.ShapeDtypeStruct((N_PAD, ROW2), jnp.float32),
            jax.ShapeDtypeStruct((N_PAD, 16), jnp.float32),
        ],
    )(acc1, b1, W2, a2, P)


def _tc_final_body(acc_ref, b2_ref, out_ref):
    u = acc_ref[0, :, 0:40] + acc_ref[1, :, 0:40]
    den = acc_ref[0, :, 40:41] + acc_ref[1, :, 40:41]
    out_ref[...] = u / (den + 1e-16) + b2_ref[...]


def _tc_final(acc2, b2):
    return pl.pallas_call(
        _tc_final_body,
        grid=(GRID,),
        in_specs=[
            pl.BlockSpec((2, BLK, ROW2), lambda i: (0, i, 0)),
            pl.BlockSpec((1, C), lambda i: (0, 0)),
        ],
        out_specs=pl.BlockSpec((BLK, C), lambda i: (i, 0)),
        out_shape=jax.ShapeDtypeStruct((N_PAD, C), jnp.float32),
    )(acc2, b2)


def _sc_edge_pass(layer, row_w):
    mesh = plsc.VectorSubcoreMesh(
        core_axis_name="c", subcore_axis_name="s", num_cores=NC)
    UNROLL = 4

    def body(table_hbm, adtab_hbm, src_hbm, dst_hbm, zeros_hbm, out_hbm,
             srcib, dstib, rows0, rows1, ob0, ob1, adr0, adr1, acc,
             is0, is1, is2, is3, id0, id1, id2, id3,
             gs0, gs1, as0, as1, ss0, ss1):
        rows = (rows0, rows1)
        obuf = (ob0, ob1)
        adr = (adr0, adr1)
        isem = (is0, is1, is2, is3)
        idsem = (id0, id1, id2, id3)
        gsem = (gs0, gs1)
        asem = (as0, as1)
        ssem = (ss0, ss1)

        cid = lax.axis_index("c")
        sid = lax.axis_index("s")
        wid = sid * NC + cid
        base = wid * CPT

        r0 = sid * ROWS_PER_TILE
        pltpu.sync_copy(zeros_hbm.at[pl.ds(r0, ROWS_PER_TILE)],
                        acc.at[pl.ds(r0, ROWS_PER_TILE)])
        plsc.subcore_barrier()

        iota = lax.iota(jnp.int32, LANES)
        hi = lax.shift_right_logical(iota, 3)

        def fire_idx(j, bi):
            pltpu.async_copy(src_hbm.at[base + j], srcib.at[bi], isem[bi])
            pltpu.async_copy(dst_hbm.at[base + j], dstib.at[bi], idsem[bi])

        def wait_idx(j, bi):
            pltpu.make_async_copy(
                src_hbm.at[base + j], srcib.at[bi], isem[bi]).wait()
            pltpu.make_async_copy(
                dst_hbm.at[base + j], dstib.at[bi], idsem[bi]).wait()

        def fire_gathers(j, b, bi):
            pltpu.async_copy(table_hbm.at[srcib.at[bi]], rows[b], gsem[b])
            pltpu.async_copy(adtab_hbm.at[dstib.at[bi]], adr[b], asem[b])

        def wait_gathers(b, bi):
            pltpu.make_async_copy(
                table_hbm.at[srcib.at[bi]], rows[b], gsem[b]).wait()
            pltpu.make_async_copy(
                adtab_hbm.at[dstib.at[bi]], adr[b], asem[b]).wait()

        def fire_scatter(b, bi):
            pltpu.async_copy(obuf[b], acc.at[dstib.at[bi]], ssem[b],
                             add=True)

        def wait_scatter(b, bi):
            pltpu.make_async_copy(
                obuf[b], acc.at[dstib.at[bi]], ssem[b]).wait()

        if layer == 1:
            def edge_one(b, k):
                as16 = rows[b][k, pl.ds(64, LANES)]
                ad16 = adr[b][k, :]
                e = as16 + ad16
                s = jnp.exp(jnp.maximum(e, 0.2 * e))
                obuf[b][k, pl.ds(64, LANES)] = s
                for v in range(4):
                    m = jnp.take_along_axis(s, hi + (2 * v), axis=0)
                    obuf[b][k, pl.ds(16 * v, LANES)] = (
                        rows[b][k, pl.ds(16 * v, LANES)] * m)
        else:
            def edge_one(b, k):
                r2 = rows[b][k, pl.ds(32, LANES)]
                adv = adr[b][k, :]
                as_b = jnp.take_along_axis(
                    r2, jnp.full((LANES,), 8, jnp.int32), axis=0)
                ad_b = jnp.take_along_axis(
                    adv, jnp.zeros((LANES,), jnp.int32), axis=0)
                e = as_b + ad_b
                s = jnp.exp(jnp.maximum(e, 0.2 * e))
                obuf[b][k, pl.ds(0, LANES)] = rows[b][k, pl.ds(0, LANES)] * s
                obuf[b][k, pl.ds(16, LANES)] = rows[b][k, pl.ds(16, LANES)] * s
                obuf[b][k, pl.ds(32, LANES)] = jnp.where(iota == 8, s, r2 * s)

        def compute(b):
            def edge_body(i, _):
                for uu in range(UNROLL):
                    edge_one(b, i * UNROLL + uu)
                return ()
            lax.fori_loop(0, CH // UNROLL, edge_body, ())

        fire_idx(0, 0)
        fire_idx(1, 1)
        wait_idx(0, 0)
        fire_gathers(0, 0, 0)

        def step(t, _):
            for u in range(4):
                j = t * 4 + u
                b = u % 2
                bi = u
                bn = (u + 1) % 2
                bin_ = (u + 1) % 4
                wait_gathers(b, bi)

                @pl.when(j >= 2)
                def _():
                    wait_scatter(b, (u + 2) % 4)

                @pl.when(j + 2 < CPT)
                def _():
                    fire_idx(j + 2, (u + 2) % 4)

                @pl.when(j + 1 < CPT)
                def _():
                    wait_idx(j + 1, bin_)
                    fire_gathers(j + 1, bn, bin_)

                compute(b)
                fire_scatter(b, bi)
            return ()

        lax.fori_loop(0, CPT // 4, step, ())

        wait_scatter(0, (CPT - 2) % 4)
        wait_scatter(1, (CPT - 1) % 4)
        plsc.subcore_barrier()
        pltpu.sync_copy(acc.at[pl.ds(r0, ROWS_PER_TILE)],
                        out_hbm.at[cid, pl.ds(r0, ROWS_PER_TILE)])

    return pl.kernel(
        body,
        out_type=jax.ShapeDtypeStruct((NC, N_PAD, row_w), jnp.float32),
        mesh=mesh,
        compiler_params=pltpu.CompilerParams(use_tc_tiling_on_sc=False),
        scratch_types=[
            pltpu.VMEM((4, CH), jnp.int32),
            pltpu.VMEM((4, CH), jnp.int32),
            pltpu.VMEM((CH, row_w), jnp.float32),
            pltpu.VMEM((CH, row_w), jnp.float32),
            pltpu.VMEM((CH, row_w), jnp.float32),
            pltpu.VMEM((CH, row_w), jnp.float32),
            pltpu.VMEM((CH, 16), jnp.float32),
            pltpu.VMEM((CH, 16), jnp.float32),
            pltpu.VMEM_SHARED((N_PAD, row_w), jnp.float32),
        ] + [pltpu.SemaphoreType.DMA] * 14,
    )


def kernel(x, edge_index, W1, a_src1, a_dst1, b1, W2, a_src2, a_dst2, b2, Q,
           epoch):
    src = edge_index[0]
    dst = edge_index[1]
    pad = E_PAD - E
    src = jnp.concatenate([src, jnp.full((pad,), PAD_DST, jnp.int32)])
    dst = jnp.concatenate([dst, jnp.full((pad,), PAD_DST, jnp.int32)])
    src2d = src.reshape(E_PAD // CH, CH)
    dst2d = dst.reshape(E_PAD // CH, CH)
    xp = jnp.concatenate([x, jnp.zeros((N_PAD - N, F_IN), jnp.float32)])

    eye = jnp.eye(HEADS, dtype=jnp.float32)
    Asrc1 = (a_src1[:, :, None] * eye[:, None, :]).reshape(HEADS * HID, HEADS)
    Adst1 = (a_dst1[:, :, None] * eye[:, None, :]).reshape(HEADS * HID, HEADS)
    A1 = jnp.concatenate([Asrc1, Adst1], axis=1)
    a2 = jnp.concatenate([a_src2.T, a_dst2.T], axis=1)
    P = jnp.repeat(eye, HID, axis=1)

    table1, adtab1 = _tc_prep1(xp, W1, A1)
    zeros1 = jnp.zeros((N_PAD, ROW1), jnp.float32)
    acc1 = _sc_edge_pass(1, ROW1)(table1, adtab1, src2d, dst2d, zeros1)
    table2, adtab2 = _tc_mid(acc1, b1.reshape(1, -1), W2, a2, P)
    zeros2 = jnp.zeros((N_PAD, ROW2), jnp.float32)
    acc2 = _sc_edge_pass(2, ROW2)(table2, adtab2, src2d, dst2d, zeros2)
    out = _tc_final(acc2, b2.reshape(1, -1))
    return (out[:N], Q)
